# Initial kernel scaffold; baseline (speedup 1.0000x reference)
import jax
import jax.numpy as jnp
from jax.experimental import pallas as pl


def kernel(type_ids, inst_ids, field_ids, values, padding_mask, type_emb, inst_pos, field_emb, value_emb):
    raise NotImplementedError("write your pallas kernel here")


# speedup vs baseline: 4.6633x; 4.6633x over previous
import functools

import jax
import jax.numpy as jnp
from jax import lax
from jax.experimental import pallas as pl
from jax.experimental.pallas import tpu as pltpu
from jax.experimental.pallas import tpu_sc as plsc

B, N, D = 4, 4096, 2048
NUM_BRANCHES, F, MAX_INST = 8, 8, 512
TOK = B * N
NC, NS, L = 2, 16, 16
NW = NC * NS
TPW = TOK // NW
K = 16
NCHUNK = TPW // K
NROWS = NUM_BRANCHES * F
TABR = NROWS + 8
IPR = MAX_INST + 8


def _prep_body(te_ref, fe_ref, ve_ref, tab_ref):
    a = te_ref[...] + fe_ref[...]
    top = jnp.concatenate([a, ve_ref[...]], axis=1)
    pad = jnp.zeros((TABR - NROWS, 2 * D), jnp.float32)
    tab_ref[...] = jnp.concatenate([top, pad], axis=0)


def _sc_body(tab_hbm, ip_hbm, t_hbm, f_hbm, i_hbm, p_hbm, v_hbm, out_hbm,
             t_v, f_v, i_v, p_v, vals_v, cidx_v, iidx_v, av_v, acc_v,
             sem_a, sem_b):
    wid = lax.axis_index("s") * NC + lax.axis_index("c")

    pltpu.sync_copy(t_hbm.at[wid], t_v)
    pltpu.sync_copy(f_hbm.at[wid], f_v)
    pltpu.sync_copy(i_hbm.at[wid], i_v)
    pltpu.sync_copy(p_hbm.at[wid], p_v)
    pltpu.sync_copy(v_hbm.at[wid], vals_v)

    for kk in range(TPW // L):
        sl = pl.ds(kk * L, L)
        t = t_v[sl]
        f = f_v[sl]
        ii = i_v[sl]
        p = p_v[sl]
        loc = jnp.clip(f - t * F, 0, F - 1)
        c = t * F + loc
        masked = p != 0
        cidx_v[kk] = jnp.where(masked, NROWS, c)
        iidx_v[kk] = jnp.where(masked, MAX_INST, ii)

    base = wid * TPW
    lane = lax.iota(jnp.int32, L)

    def chunk(g):
        cp_a = pltpu.async_copy(tab_hbm.at[cidx_v.at[g]], av_v, sem_a)
        cp_b = pltpu.async_copy(ip_hbm.at[iidx_v.at[g]], acc_v, sem_b)
        cp_a.wait()
        cp_b.wait()
        v_row = vals_v[pl.ds(g * K, K)]
        for j in range(K):
            vj = jnp.sum(jnp.where(lane == j, v_row, 0.0))
            vv = jnp.full((L,), vj, jnp.float32)

            def fma(k):
                s = pl.ds(k * L, L)
                s2 = pl.ds(D + k * L, L)
                acc_v[j, s] = acc_v[j, s] + av_v[j, s] + vv * av_v[j, s2]

            pl.loop(0, D // L)(fma)
        pltpu.sync_copy(acc_v, out_hbm.at[pl.ds(base + g * K, K)])

    pl.loop(0, NCHUNK)(chunk)


@jax.jit
def _run(type_ids, inst_ids, field_ids, values, padding_mask,
         type_emb, inst_pos, field_emb, value_emb):
    te64 = jnp.repeat(type_emb, F, axis=0)
    fe = field_emb.reshape(NROWS, D)
    ve = value_emb.reshape(NROWS, D)

    tab = pl.pallas_call(
        _prep_body,
        out_shape=jax.ShapeDtypeStruct((TABR, 2 * D), jnp.float32),
    )(te64, fe, ve)

    ip = jnp.pad(inst_pos, ((0, IPR - MAX_INST), (0, 0)))

    t2 = type_ids.reshape(NW, TPW).astype(jnp.int32)
    f2 = field_ids.reshape(NW, TPW).astype(jnp.int32)
    i2 = inst_ids.reshape(NW, TPW).astype(jnp.int32)
    p2 = padding_mask.reshape(NW, TPW).astype(jnp.int32)
    v2 = values.reshape(NW, TPW)

    mesh = plsc.VectorSubcoreMesh(core_axis_name="c", subcore_axis_name="s")
    out = pl.kernel(
        _sc_body,
        out_type=jax.ShapeDtypeStruct((TOK, D), jnp.float32),
        mesh=mesh,
        compiler_params=pltpu.CompilerParams(needs_layout_passes=False),
        scratch_types=[
            pltpu.VMEM((TPW,), jnp.int32),
            pltpu.VMEM((TPW,), jnp.int32),
            pltpu.VMEM((TPW,), jnp.int32),
            pltpu.VMEM((TPW,), jnp.int32),
            pltpu.VMEM((TPW,), jnp.float32),
            pltpu.VMEM((NCHUNK, K), jnp.int32),
            pltpu.VMEM((NCHUNK, K), jnp.int32),
            pltpu.VMEM((K, 2 * D), jnp.float32),
            pltpu.VMEM((K, D), jnp.float32),
            pltpu.SemaphoreType.DMA,
            pltpu.SemaphoreType.DMA,
        ],
    )(tab, ip, t2, f2, i2, p2, v2)
    return out.reshape(B, N, D)


def kernel(type_ids, inst_ids, field_ids, values, padding_mask,
           type_emb, inst_pos, field_emb, value_emb):
    return _run(type_ids, inst_ids, field_ids, values, padding_mask,
                type_emb, inst_pos, field_emb, value_emb)
